# baseline (speedup 1.0000x reference)
import functools

import jax
import jax.numpy as jnp
from jax import lax
from jax.experimental import pallas as pl
from jax.experimental.pallas import tpu as pltpu
from jax.experimental.pallas import tpu_sc as plsc

NC, NS = 2, 16
NW = NC * NS

B, H, D = 4096, 50, 128
ROWS = B * H
RPW = ROWS // NW
CHUNK = 128
NCH = RPW // CHUNK
NBUF = 5

_mesh = plsc.VectorSubcoreMesh(
    core_axis_name="c", subcore_axis_name="s", num_cores=NC, num_subcores=NS
)


@functools.partial(
    pl.kernel,
    out_type=jax.ShapeDtypeStruct((ROWS, D), jnp.float32),
    mesh=_mesh,
    compiler_params=pltpu.CompilerParams(needs_layout_passes=False),
    scratch_types=[
        pltpu.VMEM((RPW,), jnp.int32),
        pltpu.VMEM((NBUF, CHUNK, D), jnp.float32),
        [pltpu.SemaphoreType.DMA] * NBUF,
        [pltpu.SemaphoreType.DMA] * NBUF,
    ],
)
def _row_gather(tidx_hbm, z_hbm, out_hbm, idx_v, in_v, sem_g, sem_o):
    wid = lax.axis_index("s") * NC + lax.axis_index("c")
    jbase = wid * RPW
    pltpu.sync_copy(tidx_hbm.at[pl.ds(jbase, RPW)], idx_v)

    def gather_start(ci, b):
        pltpu.make_async_copy(
            z_hbm.at[idx_v.at[pl.ds(ci * CHUNK, CHUNK)]], in_v.at[b], sem_g[b]
        ).start()

    def gather_wait(b):
        pltpu.make_async_copy(
            z_hbm.at[idx_v.at[pl.ds(0, CHUNK)]], in_v.at[b], sem_g[b]
        ).wait()

    def out_start(ci, b):
        pltpu.make_async_copy(
            in_v.at[b],
            out_hbm.at[pl.ds(jbase + ci * CHUNK, CHUNK), :],
            sem_o[b],
        ).start()

    def out_wait(b):
        pltpu.make_async_copy(
            in_v.at[b], out_hbm.at[pl.ds(0, CHUNK), :], sem_o[b]
        ).wait()

    AHEAD = 3
    for b in range(AHEAD):
        gather_start(b, b)

    def ring_body(i, carry):
        for b in range(NBUF):
            ci = i * NBUF + b
            nb = (b + AHEAD) % NBUF

            @pl.when(ci + AHEAD < NCH)
            def _():
                @pl.when(ci >= NBUF - AHEAD)
                def _():
                    out_wait(nb)

                gather_start(ci + AHEAD, nb)

            gather_wait(b)
            out_start(ci, b)

        return carry

    lax.fori_loop(0, NCH // NBUF, ring_body, 0)
    for b in range(NBUF):
        out_wait(b)


def kernel(inputs, z):
    tidx = inputs.T.reshape(-1).astype(jnp.int32)
    rows = _row_gather(tidx, z)
    return rows.reshape(H, B, D).transpose(1, 2, 0)


# speedup vs baseline: 1.0054x; 1.0054x over previous
import functools

import jax
import jax.numpy as jnp
from jax import lax
from jax.experimental import pallas as pl
from jax.experimental.pallas import tpu as pltpu
from jax.experimental.pallas import tpu_sc as plsc

NC, NS = 2, 16
NW = NC * NS

B, H, D = 4096, 50, 128
ROWS = B * H
RPW = ROWS // NW
CHUNK = 128
NCH = RPW // CHUNK
NBUF = 5

_mesh = plsc.VectorSubcoreMesh(
    core_axis_name="c", subcore_axis_name="s", num_cores=NC, num_subcores=NS
)


@functools.partial(
    pl.kernel,
    out_type=jax.ShapeDtypeStruct((ROWS, D), jnp.float32),
    mesh=_mesh,
    compiler_params=pltpu.CompilerParams(needs_layout_passes=False),
    scratch_types=[
        pltpu.VMEM((RPW,), jnp.int32),
        pltpu.VMEM((NBUF, CHUNK, D), jnp.float32),
        [pltpu.SemaphoreType.DMA] * NBUF,
        [pltpu.SemaphoreType.DMA] * NBUF,
    ],
)
def _row_gather(tidx_hbm, z_hbm, out_hbm, idx_v, in_v, sem_g, sem_o):
    wid = lax.axis_index("s") * NC + lax.axis_index("c")
    jbase = wid * RPW
    pltpu.sync_copy(tidx_hbm.at[pl.ds(jbase, RPW)], idx_v)

    def gather_start(ci, b):
        pltpu.make_async_copy(
            z_hbm.at[idx_v.at[pl.ds(ci * CHUNK, CHUNK)]], in_v.at[b], sem_g[b]
        ).start()

    def gather_wait(b):
        pltpu.make_async_copy(
            z_hbm.at[idx_v.at[pl.ds(0, CHUNK)]], in_v.at[b], sem_g[b]
        ).wait()

    def out_start(ci, b):
        pltpu.make_async_copy(
            in_v.at[b],
            out_hbm.at[pl.ds(jbase + ci * CHUNK, CHUNK), :],
            sem_o[b],
        ).start()

    def out_wait(b):
        pltpu.make_async_copy(
            in_v.at[b], out_hbm.at[pl.ds(0, CHUNK), :], sem_o[b]
        ).wait()

    for b in range(NBUF):
        gather_start(b, b)

    def ring_body(i, carry):
        for b in range(NBUF):
            ci = i * NBUF + b
            gather_wait(b)
            out_start(ci, b)

            @pl.when(ci + NBUF < NCH)
            def _():
                out_wait(b)
                gather_start(ci + NBUF, b)

        return carry

    lax.fori_loop(0, NCH // NBUF, ring_body, 0)
    for b in range(NBUF):
        out_wait(b)


def kernel(inputs, z):
    tidx = inputs.T.reshape(-1).astype(jnp.int32)
    rows = _row_gather(tidx, z)
    return rows.reshape(H, B, D).transpose(1, 2, 0)
